# Initial kernel scaffold; baseline (speedup 1.0000x reference)
import jax
import jax.numpy as jnp
from jax.experimental import pallas as pl


def kernel(x, edge_index, W1, b1, Wmu, bmu, Wls, bls):
    raise NotImplementedError("write your pallas kernel here")


# speedup vs baseline: 10.9508x; 10.9508x over previous
import functools

import jax
import jax.numpy as jnp
from jax import lax
from jax.experimental import pallas as pl
from jax.experimental.pallas import tpu as pltpu
from jax.experimental.pallas import tpu_sc as plsc

N = 10000
E = 320000
F = 128
O = 64

NC, NS = 2, 16
NW = NC * NS
CHUNK = 128
NCHUNKS = 2528
EPAD = NCHUNKS * CHUNK
CPW = NCHUNKS // NW
EPW = CPW * CHUNK
DUMMY = N
DEGW = 128

RPAD = 12288
RPS = RPAD // NS
BLK = 384
NBLK = RPAD // BLK

_f32 = jnp.float32
_mesh = plsc.VectorSubcoreMesh(core_axis_name="c", subcore_axis_name="s")


@functools.partial(
    pl.kernel,
    out_type=jax.ShapeDtypeStruct((NC * RPAD, DEGW), _f32),
    mesh=_mesh,
    scratch_types=[
        pltpu.VMEM((CHUNK,), jnp.int32),
        pltpu.VMEM((CHUNK, DEGW), _f32),
        pltpu.VMEM_SHARED((RPAD, DEGW), _f32),
    ],
)
def _deg_kernel(dst_hbm, ones_hbm, zrow_hbm, out_hbm, dst_v, ones_v, deg_sh):
    c = lax.axis_index("c")
    s = lax.axis_index("s")
    pltpu.sync_copy(zrow_hbm, deg_sh.at[pl.ds(s * RPS, RPS)])
    pltpu.sync_copy(ones_hbm, ones_v)
    plsc.subcore_barrier()
    base = (c * NS + s) * EPW

    @pl.loop(0, CPW)
    def _(j):
        pltpu.sync_copy(dst_hbm.at[pl.ds(base + j * CHUNK, CHUNK)], dst_v)
        pltpu.sync_copy(ones_v, deg_sh.at[dst_v], add=True)

    plsc.subcore_barrier()
    pltpu.sync_copy(deg_sh.at[pl.ds(s * RPS, RPS)],
                    out_hbm.at[pl.ds(c * RPAD + s * RPS, RPS)])


@functools.partial(
    pl.kernel,
    out_type=jax.ShapeDtypeStruct((NC * RPAD, F), _f32),
    mesh=_mesh,
    scratch_types=[
        pltpu.VMEM((CHUNK,), jnp.int32),
        pltpu.VMEM((CHUNK,), jnp.int32),
        pltpu.VMEM((CHUNK, F), _f32),
        pltpu.VMEM_SHARED((RPAD, F), _f32),
        pltpu.SemaphoreType.DMA,
    ],
)
def _agg_kernel(v_hbm, src_hbm, dst_hbm, zblk_hbm, out_hbm,
                src_v, dst_v, rows_v, acc_sh, sem):
    c = lax.axis_index("c")
    s = lax.axis_index("s")
    pltpu.sync_copy(zblk_hbm, acc_sh.at[pl.ds(s * RPS, RPS)])
    plsc.subcore_barrier()
    base = (c * NS + s) * EPW

    @pl.loop(0, CPW)
    def _(j):
        pltpu.sync_copy(src_hbm.at[pl.ds(base + j * CHUNK, CHUNK)], src_v)
        pltpu.sync_copy(dst_hbm.at[pl.ds(base + j * CHUNK, CHUNK)], dst_v)
        pltpu.async_copy(v_hbm.at[src_v], rows_v, sem).wait()
        pltpu.sync_copy(rows_v, acc_sh.at[dst_v], add=True)

    plsc.subcore_barrier()
    pltpu.sync_copy(acc_sh.at[pl.ds(s * RPS, RPS)],
                    out_hbm.at[pl.ds(c * RPAD + s * RPS, RPS)])


def _dinv_block(degT_blk):
    deg = degT_blk[:, 0:1] + degT_blk[:, 1:2] + 1.0
    return lax.rsqrt(deg)


def _prep_body(x_ref, w1_ref, degT_ref, v1_ref):
    xw = jnp.dot(x_ref[...], w1_ref[...], preferred_element_type=_f32,
                 precision=lax.Precision.HIGHEST)
    v1_ref[...] = xw * _dinv_block(degT_ref[...])


def _mid_body(s_ref, v1_ref, degT_ref, b1_ref, v2_ref):
    dinv = _dinv_block(degT_ref[...])
    pre = (s_ref[0] + s_ref[1] + v1_ref[...]) * dinv + b1_ref[...]
    v2_ref[...] = jnp.maximum(pre, 0.0) * dinv


def _out_body(t_ref, v2_ref, degT_ref, wmu_ref, bmu_ref, wls_ref, bls_ref,
              mu_ref, ls_ref):
    dinv = _dinv_block(degT_ref[...])
    q = (t_ref[0] + t_ref[1] + v2_ref[...]) * dinv
    mu_ref[...] = jnp.dot(q, wmu_ref[...], preferred_element_type=_f32,
                          precision=lax.Precision.HIGHEST) + bmu_ref[...]
    ls_ref[...] = jnp.dot(q, wls_ref[...], preferred_element_type=_f32,
                          precision=lax.Precision.HIGHEST) + bls_ref[...]


_row = lambda i: (i, 0)
_full2 = pl.BlockSpec((2, RPAD), lambda i: (0, 0))

_prep_call = pl.pallas_call(
    _prep_body,
    grid=(NBLK,),
    in_specs=[pl.BlockSpec((BLK, F), _row),
              pl.BlockSpec((F, F), lambda i: (0, 0)),
              pl.BlockSpec((BLK, 2), _row)],
    out_specs=pl.BlockSpec((BLK, F), _row),
    out_shape=jax.ShapeDtypeStruct((RPAD, F), _f32),
)

_mid_call = pl.pallas_call(
    _mid_body,
    grid=(NBLK,),
    in_specs=[pl.BlockSpec((2, BLK, F), lambda i: (0, i, 0)),
              pl.BlockSpec((BLK, F), _row),
              pl.BlockSpec((BLK, 2), _row),
              pl.BlockSpec((1, F), lambda i: (0, 0))],
    out_specs=pl.BlockSpec((BLK, F), _row),
    out_shape=jax.ShapeDtypeStruct((RPAD, F), _f32),
)

_out_call = pl.pallas_call(
    _out_body,
    grid=(NBLK,),
    in_specs=[pl.BlockSpec((2, BLK, F), lambda i: (0, i, 0)),
              pl.BlockSpec((BLK, F), _row),
              pl.BlockSpec((BLK, 2), _row),
              pl.BlockSpec((F, O), lambda i: (0, 0)),
              pl.BlockSpec((1, O), lambda i: (0, 0)),
              pl.BlockSpec((F, O), lambda i: (0, 0)),
              pl.BlockSpec((1, O), lambda i: (0, 0))],
    out_specs=[pl.BlockSpec((BLK, O), _row), pl.BlockSpec((BLK, O), _row)],
    out_shape=[jax.ShapeDtypeStruct((RPAD, O), _f32),
               jax.ShapeDtypeStruct((RPAD, O), _f32)],
)


def kernel(x, edge_index, W1, b1, Wmu, bmu, Wls, bls):
    src = edge_index[0].astype(jnp.int32)
    dst = edge_index[1].astype(jnp.int32)
    pad = EPAD - E
    fill = jnp.full((pad,), DUMMY, jnp.int32)
    src1d = jnp.concatenate([src, fill])
    dst1d = jnp.concatenate([dst, fill])

    ones = jnp.ones((CHUNK, DEGW), _f32)
    zrow = jnp.zeros((RPS, DEGW), _f32)
    zblk = jnp.zeros((RPS, F), _f32)

    degp = _deg_kernel(dst1d, ones, zrow)
    degT = degp.reshape(NC, RPAD, DEGW)[:, :, 0].T

    x_pad = jnp.concatenate([x, jnp.zeros((RPAD - N, F), _f32)])
    v1 = _prep_call(x_pad, W1, degT)
    s = _agg_kernel(v1, src1d, dst1d, zblk).reshape(NC, RPAD, F)
    v2 = _mid_call(s, v1, degT, b1.reshape(1, F))
    t = _agg_kernel(v2, src1d, dst1d, zblk).reshape(NC, RPAD, F)
    mu, ls = _out_call(t, v2, degT, Wmu, bmu.reshape(1, O),
                       Wls, bls.reshape(1, O))
    return (mu[:N], ls[:N])
